# baseline (speedup 1.0000x reference)
import functools

import jax
import jax.numpy as jnp
from jax import lax
from jax.experimental import pallas as pl
from jax.experimental.pallas import tpu as pltpu
from jax.experimental.pallas import tpu_sc as plsc

D_MODEL = 2048
S = 8192
TABLE_ROWS = 72

NC = 2
NS = 16
NW = NC * NS
B_PER_W = S // NW
CH = 16
NCH = B_PER_W // CH


def _table_body(rank_ref, file_ref, flag_ref, out_ref):
    rank_part = jnp.concatenate(
        [jnp.broadcast_to(rank_ref[k:k + 1, :], (8, D_MODEL)) for k in range(8)],
        axis=0)
    file_part = jnp.concatenate([file_ref[...]] * 8, axis=0)
    out_ref[0:64, :] = rank_part + file_part
    out_ref[64:69, :] = flag_ref[...]


KDUP = 4


def _build_table(rank_embed, file_embed, flag_pad):
    return pl.pallas_call(
        _table_body,
        grid=(KDUP,),
        out_specs=pl.BlockSpec((TABLE_ROWS, D_MODEL), lambda i: (i, 0)),
        out_shape=jax.ShapeDtypeStruct((KDUP * TABLE_ROWS, D_MODEL),
                                       jnp.float32),
    )(rank_embed, file_embed, flag_pad)


_mesh = plsc.VectorSubcoreMesh(core_axis_name="c", subcore_axis_name="s")


NBUF = 3


@functools.partial(
    pl.kernel,
    mesh=_mesh,
    out_type=jax.ShapeDtypeStruct((S, D_MODEL), jnp.float32),
    scratch_types=[
        pltpu.VMEM((NCH, CH), jnp.int32),
    ]
    + [pltpu.VMEM((CH, D_MODEL), jnp.float32) for _ in range(NBUF)]
    + [pltpu.SemaphoreType.DMA for _ in range(2 * NBUF)],
)
def _gather_kernel(idx_hbm, table_hbm, out_hbm, idx_v, *scr):
    bufs = scr[:NBUF]
    gsems = scr[NBUF:2 * NBUF]
    ssems = scr[2 * NBUF:]
    wid = lax.axis_index("s") * NC + lax.axis_index("c")
    base = wid * B_PER_W
    pltpu.sync_copy(idx_hbm.at[wid], idx_v)
    gcp = [None] * NBUF
    scp = [None] * NBUF
    for b in range(min(NBUF, NCH)):
        gcp[b] = pltpu.async_copy(table_hbm.at[idx_v.at[b]], bufs[b], gsems[b])
    for c in range(NCH):
        b = c % NBUF
        gcp[b].wait()
        scp[b] = pltpu.async_copy(bufs[b], out_hbm.at[pl.ds(base + c * CH, CH)],
                                  ssems[b])
        if c + NBUF < NCH:
            scp[b].wait()
            gcp[b] = pltpu.async_copy(
                table_hbm.at[idx_v.at[c + NBUF]], bufs[b], gsems[b])
    for c in range(max(0, NCH - NBUF), NCH):
        scp[c % NBUF].wait()


def kernel(positions, rank_embed, file_embed, flag_embed):
    positions = positions.astype(jnp.int32)
    flag_pad = flag_embed.astype(jnp.float32)
    table = _build_table(rank_embed.astype(jnp.float32),
                         file_embed.astype(jnp.float32), flag_pad)
    idx = positions.reshape(NW, NCH, CH)
    rep_off = (jnp.arange(NW, dtype=jnp.int32) % KDUP * TABLE_ROWS)
    idx = idx + rep_off[:, None, None]
    return _gather_kernel(idx, table)


# speedup vs baseline: 1.0233x; 1.0233x over previous
import functools

import jax
import jax.numpy as jnp
from jax import lax
from jax.experimental import pallas as pl
from jax.experimental.pallas import tpu as pltpu
from jax.experimental.pallas import tpu_sc as plsc

D_MODEL = 2048
S = 8192
TABLE_ROWS = 72

NC = 2
NS = 16
NW = NC * NS
B_PER_W = S // NW
CH = 16
NCH = B_PER_W // CH


def _table_body(rank_ref, file_ref, flag_ref, out_ref):
    rank_part = jnp.concatenate(
        [jnp.broadcast_to(rank_ref[k:k + 1, :], (8, D_MODEL)) for k in range(8)],
        axis=0)
    file_part = jnp.concatenate([file_ref[...]] * 8, axis=0)
    out_ref[0:64, :] = rank_part + file_part
    out_ref[64:69, :] = flag_ref[...]


KDUP = 8


def _build_table(rank_embed, file_embed, flag_pad):
    return pl.pallas_call(
        _table_body,
        grid=(KDUP,),
        out_specs=pl.BlockSpec((TABLE_ROWS, D_MODEL), lambda i: (i, 0)),
        out_shape=jax.ShapeDtypeStruct((KDUP * TABLE_ROWS, D_MODEL),
                                       jnp.float32),
    )(rank_embed, file_embed, flag_pad)


_mesh = plsc.VectorSubcoreMesh(core_axis_name="c", subcore_axis_name="s")


NBUF = 3


@functools.partial(
    pl.kernel,
    mesh=_mesh,
    out_type=jax.ShapeDtypeStruct((S, D_MODEL), jnp.float32),
    scratch_types=[
        pltpu.VMEM((NCH, CH), jnp.int32),
    ]
    + [pltpu.VMEM((CH, D_MODEL), jnp.float32) for _ in range(NBUF)]
    + [pltpu.SemaphoreType.DMA for _ in range(2 * NBUF)],
)
def _gather_kernel(idx_hbm, table_hbm, out_hbm, idx_v, *scr):
    bufs = scr[:NBUF]
    gsems = scr[NBUF:2 * NBUF]
    ssems = scr[2 * NBUF:]
    wid = lax.axis_index("s") * NC + lax.axis_index("c")
    base = wid * B_PER_W
    pltpu.sync_copy(idx_hbm.at[wid], idx_v)
    gcp = [None] * NBUF
    scp = [None] * NBUF
    for b in range(min(NBUF, NCH)):
        gcp[b] = pltpu.async_copy(table_hbm.at[idx_v.at[b]], bufs[b], gsems[b])
    for c in range(NCH):
        b = c % NBUF
        gcp[b].wait()
        scp[b] = pltpu.async_copy(bufs[b], out_hbm.at[pl.ds(base + c * CH, CH)],
                                  ssems[b])
        if c + NBUF < NCH:
            scp[b].wait()
            gcp[b] = pltpu.async_copy(
                table_hbm.at[idx_v.at[c + NBUF]], bufs[b], gsems[b])
    for c in range(max(0, NCH - NBUF), NCH):
        scp[c % NBUF].wait()


def kernel(positions, rank_embed, file_embed, flag_embed):
    positions = positions.astype(jnp.int32)
    flag_pad = flag_embed.astype(jnp.float32)
    table = _build_table(rank_embed.astype(jnp.float32),
                         file_embed.astype(jnp.float32), flag_pad)
    idx = positions.reshape(NW, NCH, CH)
    rep_off = (jnp.arange(NW, dtype=jnp.int32) % KDUP * TABLE_ROWS)
    idx = idx + rep_off[:, None, None]
    return _gather_kernel(idx, table)
